# baseline (speedup 1.0000x reference)
import functools
import jax
import jax.numpy as jnp
from jax import lax
from jax.experimental import pallas as pl
from jax.experimental.pallas import tpu as pltpu
from jax.experimental.pallas import tpu_sc as plsc

N = 10000
G = 16
IN = 128
HID = 64
OUT = 128
NEG_A = 0.01
NEG_G = 0.2

NP = 10240
E = 320000
ET = E + N
PAD_E = 344064
EROWS = PAD_E // 128
NSUB = 16
PT_ROWS = EROWS // NSUB
P1_CH = PT_ROWS // 8
NPT = NP // NSUB

_f32 = jnp.float32
_i32 = jnp.int32


def _lrelu(x, s):
    return jnp.where(x > 0, x, x * s)


def _splat(v, lane):
    idx = jnp.full((16, 1), lane, dtype=_i32)
    dn = lax.GatherDimensionNumbers(
        offset_dims=(), collapsed_slice_dims=(0,), start_index_map=(0,))
    return lax.gather(v, idx, dn, slice_sizes=(1,),
                      mode=lax.GatherScatterMode.PROMISE_IN_BOUNDS)


def _counts_body(b_ref, cnt_ref, fi_ref):
    b = b_ref[...]
    for g in range(G):
        cnt = jnp.sum((b == g).astype(_f32))
        fi = jnp.sum((b < g).astype(_i32))
        fi = jnp.minimum(fi, N - 1)
        cnt_ref[pl.ds(g, 1), :] = jnp.full((1, 128), cnt, _f32)
        fi_ref[pl.ds(g, 1), :] = jnp.full((1, 128), fi, _i32)


def _counts(batch2d):
    return pl.pallas_call(
        _counts_body,
        out_shape=(jax.ShapeDtypeStruct((G, 128), _f32),
                   jax.ShapeDtypeStruct((G, 128), _i32)),
    )(batch2d)


_R1 = 512


def _dense1_body(x_ref, we_ref, be_ref, l1_ref, aa_ref, a_ref, asad_ref):
    h0 = jnp.dot(x_ref[...], we_ref[...], preferred_element_type=_f32)
    h0 = _lrelu(h0 + be_ref[...], NEG_A)
    a = jnp.dot(h0, l1_ref[...], preferred_element_type=_f32)
    a_ref[...] = a
    h1 = a[:, :HID]
    asad_ref[...] = lax.dot_general(
        aa_ref[...], h1, (((1,), (1,)), ((), ())),
        preferred_element_type=_f32)


def _dense1(xp, W_emb, b_emb, l1cat, aa1):
    grid = NP // _R1
    return pl.pallas_call(
        _dense1_body,
        grid=(grid,),
        in_specs=[
            pl.BlockSpec((_R1, IN), lambda i: (i, 0)),
            pl.BlockSpec((IN, HID), lambda i: (0, 0)),
            pl.BlockSpec((1, HID), lambda i: (0, 0)),
            pl.BlockSpec((HID, 128), lambda i: (0, 0)),
            pl.BlockSpec((8, HID), lambda i: (0, 0)),
        ],
        out_specs=(
            pl.BlockSpec((_R1, 128), lambda i: (i, 0)),
            pl.BlockSpec((8, _R1), lambda i: (0, i)),
        ),
        out_shape=(jax.ShapeDtypeStruct((NP, 128), _f32),
                   jax.ShapeDtypeStruct((8, NP), _f32)),
    )(xp, W_emb, b_emb, l1cat, aa1)


def _dense2_body(pl_ref, ph_ref, b_ref, l2_ref, aa_ref, a_ref, asad_ref):
    s = jnp.concatenate([pl_ref[...], ph_ref[...]], axis=1)
    hin = _lrelu(s + b_ref[...], NEG_A)
    a = jnp.dot(hin, l2_ref[...], preferred_element_type=_f32)
    a_ref[...] = a
    h2 = a[:, :HID]
    asad_ref[...] = lax.dot_general(
        aa_ref[...], h2, (((1,), (1,)), ((), ())),
        preferred_element_type=_f32)


def _dense2(p_lo, p_hi, b1, l2cat, aa2):
    grid = NP // _R1
    return pl.pallas_call(
        _dense2_body,
        grid=(grid,),
        in_specs=[
            pl.BlockSpec((_R1, 32), lambda i: (i, 0)),
            pl.BlockSpec((_R1, 32), lambda i: (i, 0)),
            pl.BlockSpec((1, HID), lambda i: (0, 0)),
            pl.BlockSpec((HID, 128), lambda i: (0, 0)),
            pl.BlockSpec((8, HID), lambda i: (0, 0)),
        ],
        out_specs=(
            pl.BlockSpec((_R1, 128), lambda i: (i, 0)),
            pl.BlockSpec((8, _R1), lambda i: (0, i)),
        ),
        out_shape=(jax.ShapeDtypeStruct((NP, 128), _f32),
                   jax.ShapeDtypeStruct((8, NP), _f32)),
    )(p_lo, p_hi, b1, l2cat, aa2)


def _readout_body(pl_ref, ph_ref, b2_ref, wh_ref, bh_ref, o_ref):
    r = jnp.concatenate([pl_ref[...], ph_ref[...]], axis=1)
    r = _lrelu(r + b2_ref[...], NEG_A)
    o_ref[...] = jnp.dot(r, wh_ref[...], preferred_element_type=_f32) \
        + bh_ref[...]


def _readout(p_lo, p_hi, b2, W_head, b_head):
    return pl.pallas_call(
        _readout_body,
        out_shape=jax.ShapeDtypeStruct((G, OUT), _f32),
    )(p_lo, p_hi, b2, W_head, b_head)


def _edge_body(a4, asad, src2, dst2, zrows, zvec, out_hbm,
               as_l, ad_l, den_sl, src_c, dst_c, idx_c, e_c, rows,
               den_sp, out_sp, lsem, dsem, gsem, ssem):
    cid = lax.axis_index("c")
    sid = lax.axis_index("s")

    pltpu.sync_copy(zrows.at[pl.ds(sid * NPT, NPT), :],
                    out_sp.at[pl.ds(sid * NPT, NPT), :])
    pltpu.sync_copy(zvec.at[pl.ds(sid * NPT, NPT)],
                    den_sp.at[pl.ds(sid * NPT, NPT)])
    pltpu.sync_copy(asad.at[0], as_l)
    pltpu.sync_copy(asad.at[1], ad_l)
    plsc.subcore_barrier()

    row_base = sid * PT_ROWS

    pltpu.async_copy(src2.at[pl.ds(row_base, 8), :], src_c.at[0], lsem)
    pltpu.async_copy(dst2.at[pl.ds(row_base, 8), :], dst_c.at[0], lsem)

    def chunk(i, carry):
        p = i % 2
        pltpu.make_async_copy(src2.at[pl.ds(row_base, 8), :],
                              src_c.at[p], lsem).wait()
        pltpu.make_async_copy(dst2.at[pl.ds(row_base, 8), :],
                              dst_c.at[p], lsem).wait()
        for q in range(8):
            for l in range(8):
                sv = src_c[p, q, pl.ds(l * 16, 16)]
                idx_c[p, q, pl.ds(l * 16, 16)] = sv * 4 + cid
        @pl.when(i >= 1)
        def _():
            for q in range(8):
                pltpu.make_async_copy(
                    e_c.at[0, 0], den_sp.at[pl.ds(0, 128)], dsem).wait()
                pltpu.make_async_copy(
                    rows.at[0, pl.ds(0, 128), :],
                    out_sp.at[pl.ds(0, 128), :], ssem).wait()
        gd = []
        for q in range(8):
            gd.append(pltpu.async_copy(
                a4.at[idx_c.at[p, q]],
                rows.at[p, pl.ds(q * 128, 128), :], gsem))
        @pl.when(i < P1_CH - 1)
        def _():
            r1 = row_base + (i + 1) * 8
            pn = (i + 1) % 2
            pltpu.async_copy(src2.at[pl.ds(r1, 8), :], src_c.at[pn], lsem)
            pltpu.async_copy(dst2.at[pl.ds(r1, 8), :], dst_c.at[pn], lsem)
        for q in range(8):
            for l in range(8):
                sv = src_c[p, q, pl.ds(l * 16, 16)]
                dv = dst_c[p, q, pl.ds(l * 16, 16)]
                al = plsc.load_gather(as_l, [sv]) \
                    + plsc.load_gather(ad_l, [dv])
                e_c[p, q, pl.ds(l * 16, 16)] = jnp.exp(_lrelu(al, NEG_G))
        for q in range(8):
            pltpu.async_copy(e_c.at[p, q], den_sp.at[dst_c.at[p, q]],
                             dsem, add=True)
        for d in gd:
            d.wait()
        for q in range(8):
            for l in range(8):
                ev = e_c[p, q, pl.ds(l * 16, 16)]
                for t in range(16):
                    ei = q * 128 + l * 16 + t
                    es = _splat(ev, t)
                    rows[p, ei, pl.ds(0, 16)] = \
                        rows[p, ei, pl.ds(0, 16)] * es
                    rows[p, ei, pl.ds(16, 16)] = \
                        rows[p, ei, pl.ds(16, 16)] * es
        for q in range(8):
            pltpu.async_copy(rows.at[p, pl.ds(q * 128, 128), :],
                             out_sp.at[dst_c.at[p, q]], ssem, add=True)
        return carry

    lax.fori_loop(0, P1_CH, chunk, 0)
    for q in range(8):
        pltpu.make_async_copy(
            e_c.at[0, 0], den_sp.at[pl.ds(0, 128)], dsem).wait()
        pltpu.make_async_copy(
            rows.at[0, pl.ds(0, 128), :],
            out_sp.at[pl.ds(0, 128), :], ssem).wait()
    plsc.subcore_barrier()

    n0 = sid * NPT
    pltpu.sync_copy(out_sp.at[pl.ds(n0, NPT), :],
                    rows.at[0, pl.ds(0, NPT), :])
    pltpu.sync_copy(den_sp.at[pl.ds(n0, NPT)], den_sl)

    def ngroup(g, carry):
        dv = den_sl[pl.ds(g * 16, 16)]
        inv = 1.0 / (dv + 1e-16)
        for t in range(16):
            iv = _splat(inv, t)
            ri = g * 16 + t
            rows[0, ri, pl.ds(0, 16)] = rows[0, ri, pl.ds(0, 16)] * iv
            rows[0, ri, pl.ds(16, 16)] = rows[0, ri, pl.ds(16, 16)] * iv
        return carry

    lax.fori_loop(0, NPT // 16, ngroup, 0)
    pltpu.sync_copy(rows.at[0, pl.ds(0, NPT), :],
                    out_hbm.at[cid, pl.ds(n0, NPT), :])


@functools.cache
def _edge_layer_fn():
  return functools.partial(
    pl.kernel,
    out_type=jax.ShapeDtypeStruct((2, NP, 32), _f32),
    mesh=plsc.VectorSubcoreMesh(core_axis_name="c", subcore_axis_name="s",
                                num_cores=2, num_subcores=NSUB),
    scratch_types=[
        pltpu.VMEM((NP,), _f32),
        pltpu.VMEM((NP,), _f32),
        pltpu.VMEM((NPT,), _f32),
        pltpu.VMEM((2, 8, 128), _i32),
        pltpu.VMEM((2, 8, 128), _i32),
        pltpu.VMEM((2, 8, 128), _i32),
        pltpu.VMEM((2, 8, 128), _f32),
        pltpu.VMEM((2, 1024, 32), _f32),
        pltpu.VMEM_SHARED((NP,), _f32),
        pltpu.VMEM_SHARED((NP, 32), _f32),
        pltpu.SemaphoreType.DMA,
        pltpu.SemaphoreType.DMA,
        pltpu.SemaphoreType.DMA,
        pltpu.SemaphoreType.DMA,
    ],
    compiler_params=pltpu.CompilerParams(needs_layout_passes=False,
                                         use_tc_tiling_on_sc=False),
  )(_edge_body)


def _edge_layer(a4, asad, src2, dst2, zrows, zvec):
    return _edge_layer_fn()(a4, asad, src2, dst2, zrows, zvec)


L2CAP = PT_ROWS * 128


def _edge2_body(a4, asad, src2, dst2, fi_hbm, out_hbm,
                as_l, ad_l, slot_l, den_l, fi_v, sg_v,
                src_c, dst_c, hsrc, hslot, he, hw_idx, rows16,
                den_sp, acc_sp, lsem, dsem, gsem, ssem):
    cid = lax.axis_index("c")
    sid = lax.axis_index("s")

    def memset_chunk(i, carry):
        slot_l[pl.ds(i * 16, 16)] = jnp.full((16,), 16, _i32)
        return carry
    lax.fori_loop(0, NP // 16, memset_chunk, 0)
    fi_v[pl.ds(16, 16)] = jnp.zeros((16,), _i32)
    pltpu.sync_copy(fi_hbm, fi_v.at[pl.ds(0, 16)])
    lanes = lax.iota(_i32, 16)
    plsc.store_scatter(slot_l, [fi_v[pl.ds(0, 16)]], lanes)

    def hfill_chunk(i, carry):
        hsrc[pl.ds(i * 16, 16)] = jnp.zeros((16,), _i32)
        hslot[pl.ds(i * 16, 16)] = jnp.full((16,), 16, _i32)
        return carry
    lax.fori_loop(0, L2CAP // 16, hfill_chunk, 0)

    @pl.when(sid == 0)
    def _():
        z16 = jnp.zeros((16,), _f32)
        den_l[pl.ds(0, 16)] = z16
        den_l[pl.ds(16, 16)] = z16
        pltpu.sync_copy(den_l.at[pl.ds(0, 32)], den_sp)
        for r in range(16):
            rows16[r, pl.ds(0, 16)] = z16
            rows16[r, pl.ds(16, 16)] = z16
        pltpu.sync_copy(rows16, acc_sp.at[pl.ds(0, 16), :])
        pltpu.sync_copy(rows16, acc_sp.at[pl.ds(16, 16), :])

    pltpu.sync_copy(asad.at[0], as_l)
    pltpu.sync_copy(asad.at[1], ad_l)
    plsc.subcore_barrier()

    row_base = sid * PT_ROWS
    pltpu.async_copy(src2.at[pl.ds(row_base, 8), :], src_c.at[0], lsem)
    pltpu.async_copy(dst2.at[pl.ds(row_base, 8), :], dst_c.at[0], lsem)

    def p1_chunk(i, cnt):
        p = i % 2
        pltpu.make_async_copy(src2.at[pl.ds(row_base, 8), :],
                              src_c.at[p], lsem).wait()
        pltpu.make_async_copy(dst2.at[pl.ds(row_base, 8), :],
                              dst_c.at[p], lsem).wait()
        @pl.when(i < P1_CH - 1)
        def _():
            r1 = row_base + (i + 1) * 8
            pn = (i + 1) % 2
            pltpu.async_copy(src2.at[pl.ds(r1, 8), :], src_c.at[pn], lsem)
            pltpu.async_copy(dst2.at[pl.ds(r1, 8), :], dst_c.at[pn], lsem)
        for q in range(8):
            for l in range(8):
                sv = src_c[p, q, pl.ds(l * 16, 16)]
                dv = dst_c[p, q, pl.ds(l * 16, 16)]
                slv = plsc.load_gather(slot_l, [dv])
                hit = slv < 16
                plsc.store_compressed(hsrc.at[pl.ds(cnt, 16)], sv,
                                      mask=hit)
                plsc.store_compressed(hslot.at[pl.ds(cnt, 16)], slv,
                                      mask=hit)
                cnt = cnt + jnp.sum(hit.astype(_i32))
        return cnt

    cnt = lax.fori_loop(0, P1_CH, p1_chunk, 0)
    plsc.subcore_barrier()

    ngroups = (cnt + 15) >> 4

    def p2_group(g, carry):
        b = g * 16
        sv = hsrc[pl.ds(b, 16)]
        slv = hslot[pl.ds(b, 16)]
        hw_idx[0, pl.ds(0, 16)] = sv * 4 + cid
        gd = pltpu.async_copy(a4.at[hw_idx.at[0]], rows16, gsem)
        dvv = plsc.load_gather(fi_v, [slv])
        al = plsc.load_gather(as_l, [sv]) + plsc.load_gather(ad_l, [dvv])
        ev = jnp.exp(_lrelu(al, NEG_G))
        he[pl.ds(0, 16)] = ev
        hw_idx[1, pl.ds(0, 16)] = slv
        pltpu.async_copy(he.at[pl.ds(0, 16)],
                         den_sp.at[hw_idx.at[1]], dsem, add=True).wait()
        gd.wait()
        for t in range(16):
            es = _splat(ev, t)
            rows16[t, pl.ds(0, 16)] = rows16[t, pl.ds(0, 16)] * es
            rows16[t, pl.ds(16, 16)] = rows16[t, pl.ds(16, 16)] * es
        pltpu.async_copy(rows16, acc_sp.at[hw_idx.at[1]],
                         ssem, add=True).wait()
        return carry

    lax.fori_loop(0, ngroups, p2_group, 0)
    plsc.subcore_barrier()

    @pl.when(sid == 0)
    def _():
        pltpu.sync_copy(den_sp, den_l.at[pl.ds(0, 32)])
        sg = plsc.load_gather(slot_l, [fi_v[pl.ds(0, 16)]])
        sg_v[pl.ds(0, 16)] = sg
        pltpu.async_copy(acc_sp.at[sg_v], rows16, gsem).wait()
        inv = 1.0 / (plsc.load_gather(den_l, [sg]) + 1e-16)
        for t in range(16):
            iv = _splat(inv, t)
            rows16[t, pl.ds(0, 16)] = rows16[t, pl.ds(0, 16)] * iv
            rows16[t, pl.ds(16, 16)] = rows16[t, pl.ds(16, 16)] * iv
        pltpu.sync_copy(rows16, out_hbm.at[cid])


@functools.cache
def _edge_layer2_fn():
  return functools.partial(
    pl.kernel,
    out_type=jax.ShapeDtypeStruct((2, G, 32), _f32),
    mesh=plsc.VectorSubcoreMesh(core_axis_name="c", subcore_axis_name="s",
                                num_cores=2, num_subcores=NSUB),
    scratch_types=[
        pltpu.VMEM((NP,), _f32),
        pltpu.VMEM((NP,), _f32),
        pltpu.VMEM((NP,), _i32),
        pltpu.VMEM((32,), _f32),
        pltpu.VMEM((32,), _i32),
        pltpu.VMEM((16,), _i32),
        pltpu.VMEM((2, 8, 128), _i32),
        pltpu.VMEM((2, 8, 128), _i32),
        pltpu.VMEM((L2CAP,), _i32),
        pltpu.VMEM((L2CAP,), _i32),
        pltpu.VMEM((16,), _f32),
        pltpu.VMEM((2, 16), _i32),
        pltpu.VMEM((16, 32), _f32),
        pltpu.VMEM_SHARED((32,), _f32),
        pltpu.VMEM_SHARED((32, 32), _f32),
        pltpu.SemaphoreType.DMA,
        pltpu.SemaphoreType.DMA,
        pltpu.SemaphoreType.DMA,
        pltpu.SemaphoreType.DMA,
    ],
    compiler_params=pltpu.CompilerParams(needs_layout_passes=False,
                                         use_tc_tiling_on_sc=False),
  )(_edge2_body)


def _edge_layer2(a4, asad, src2, dst2, fi16):
    return _edge_layer2_fn()(a4, asad, src2, dst2, fi16)


def kernel(x, edge_index, batch, W_emb, b_emb, lin1, a1s, a1d, b1,
           lin2, a2s, a2d, b2, W_head, b_head):
    xp = jnp.pad(x, ((0, NP - N), (0, 0)))
    batch_p = jnp.pad(batch, (0, NP - N), constant_values=G)

    loop = jnp.arange(N, dtype=_i32)
    npad = PAD_E - ET
    src = jnp.concatenate(
        [edge_index[0], loop, jnp.zeros((npad,), _i32)])
    dst = jnp.concatenate(
        [edge_index[1], loop,
         N + (jnp.arange(npad, dtype=_i32) % (NP - N))])
    src2 = src.reshape(EROWS, 128)
    dst2 = dst.reshape(EROWS, 128)

    z6 = jnp.zeros((6, HID), _f32)
    l1cat = jnp.concatenate(
        [lin1, a1s[:, None], a1d[:, None], jnp.zeros((HID, 62), _f32)], 1)
    l2cat = jnp.concatenate(
        [lin2, a2s[:, None], a2d[:, None], jnp.zeros((HID, 62), _f32)], 1)
    aa1 = jnp.concatenate([a1s[None], a1d[None], z6], 0)
    aa2 = jnp.concatenate([a2s[None], a2d[None], z6], 0)

    zrows = jnp.zeros((NP, 32), _f32)
    zvec = jnp.zeros((NP,), _f32)

    cnt, fi = _counts(batch_p.reshape(80, 128))

    a1_, asad1 = _dense1(xp, W_emb, b_emb.reshape(1, HID), l1cat, aa1)
    p1 = _edge_layer(a1_.reshape(NP * 4, 32), asad1, src2, dst2, zrows, zvec)

    a2_, asad2 = _dense2(p1[0], p1[1], b1.reshape(1, HID), l2cat, aa2)
    p2 = _edge_layer2(a2_.reshape(NP * 4, 32), asad2, src2, dst2, fi[:, 0])

    out = _readout(p2[0], p2[1], b2.reshape(1, HID),
                   W_head, b_head.reshape(1, OUT))
    num_nodes = cnt[:, :1]
    return (out, num_nodes)


# speedup vs baseline: 1.3265x; 1.3265x over previous
import functools
import jax
import jax.numpy as jnp
from jax import lax
from jax.experimental import pallas as pl
from jax.experimental.pallas import tpu as pltpu
from jax.experimental.pallas import tpu_sc as plsc

N = 10000
G = 16
IN = 128
HID = 64
OUT = 128
NEG_A = 0.01
NEG_G = 0.2

NP = 10240
E = 320000
ET = E + N
PAD_E = 335872
EROWS = PAD_E // 128
NSUB = 16
PT_ROWS = EROWS // NSUB
P1_CH = PT_ROWS // 4
NPT = NP // NSUB

_f32 = jnp.float32
_i32 = jnp.int32


def _lrelu(x, s):
    return jnp.where(x > 0, x, x * s)


def _splat(v, lane):
    idx = jnp.full((16, 1), lane, dtype=_i32)
    dn = lax.GatherDimensionNumbers(
        offset_dims=(), collapsed_slice_dims=(0,), start_index_map=(0,))
    return lax.gather(v, idx, dn, slice_sizes=(1,),
                      mode=lax.GatherScatterMode.PROMISE_IN_BOUNDS)


def _counts_body(b_ref, cnt_ref, fi_ref):
    b = b_ref[...]
    for g in range(G):
        cnt = jnp.sum((b == g).astype(_f32))
        fi = jnp.sum((b < g).astype(_i32))
        fi = jnp.minimum(fi, N - 1)
        cnt_ref[pl.ds(g, 1), :] = jnp.full((1, 128), cnt, _f32)
        fi_ref[pl.ds(g, 1), :] = jnp.full((1, 128), fi, _i32)


def _counts(batch2d):
    return pl.pallas_call(
        _counts_body,
        out_shape=(jax.ShapeDtypeStruct((G, 128), _f32),
                   jax.ShapeDtypeStruct((G, 128), _i32)),
    )(batch2d)


_R1 = 512


def _dense1_body(x_ref, we_ref, be_ref, l1_ref, aa_ref, a_ref, asad_ref):
    h0 = jnp.dot(x_ref[...], we_ref[...], preferred_element_type=_f32)
    h0 = _lrelu(h0 + be_ref[...], NEG_A)
    a = jnp.dot(h0, l1_ref[...], preferred_element_type=_f32)
    a_ref[...] = a
    h1 = a[:, :HID]
    asad_ref[...] = lax.dot_general(
        aa_ref[...], h1, (((1,), (1,)), ((), ())),
        preferred_element_type=_f32)


def _dense1(xp, W_emb, b_emb, l1cat, aa1):
    grid = NP // _R1
    return pl.pallas_call(
        _dense1_body,
        grid=(grid,),
        in_specs=[
            pl.BlockSpec((_R1, IN), lambda i: (i, 0)),
            pl.BlockSpec((IN, HID), lambda i: (0, 0)),
            pl.BlockSpec((1, HID), lambda i: (0, 0)),
            pl.BlockSpec((HID, 128), lambda i: (0, 0)),
            pl.BlockSpec((8, HID), lambda i: (0, 0)),
        ],
        out_specs=(
            pl.BlockSpec((_R1, 128), lambda i: (i, 0)),
            pl.BlockSpec((8, _R1), lambda i: (0, i)),
        ),
        out_shape=(jax.ShapeDtypeStruct((NP, 128), _f32),
                   jax.ShapeDtypeStruct((8, NP), _f32)),
    )(xp, W_emb, b_emb, l1cat, aa1)


def _dense2_body(pl_ref, ph_ref, b_ref, l2_ref, aa_ref, a_ref, asad_ref):
    s = jnp.concatenate([pl_ref[...], ph_ref[...]], axis=1)
    hin = _lrelu(s + b_ref[...], NEG_A)
    a = jnp.dot(hin, l2_ref[...], preferred_element_type=_f32)
    a_ref[...] = a
    h2 = a[:, :HID]
    asad_ref[...] = lax.dot_general(
        aa_ref[...], h2, (((1,), (1,)), ((), ())),
        preferred_element_type=_f32)


def _dense2(p_lo, p_hi, b1, l2cat, aa2):
    grid = NP // _R1
    return pl.pallas_call(
        _dense2_body,
        grid=(grid,),
        in_specs=[
            pl.BlockSpec((_R1, 32), lambda i: (i, 0)),
            pl.BlockSpec((_R1, 32), lambda i: (i, 0)),
            pl.BlockSpec((1, HID), lambda i: (0, 0)),
            pl.BlockSpec((HID, 128), lambda i: (0, 0)),
            pl.BlockSpec((8, HID), lambda i: (0, 0)),
        ],
        out_specs=(
            pl.BlockSpec((_R1, 128), lambda i: (i, 0)),
            pl.BlockSpec((8, _R1), lambda i: (0, i)),
        ),
        out_shape=(jax.ShapeDtypeStruct((NP, 128), _f32),
                   jax.ShapeDtypeStruct((8, NP), _f32)),
    )(p_lo, p_hi, b1, l2cat, aa2)


def _readout_body(pl_ref, ph_ref, b2_ref, wh_ref, bh_ref, o_ref):
    r = jnp.concatenate([pl_ref[...], ph_ref[...]], axis=1)
    r = _lrelu(r + b2_ref[...], NEG_A)
    o_ref[...] = jnp.dot(r, wh_ref[...], preferred_element_type=_f32) \
        + bh_ref[...]


def _readout(p_lo, p_hi, b2, W_head, b_head):
    return pl.pallas_call(
        _readout_body,
        out_shape=jax.ShapeDtypeStruct((G, OUT), _f32),
    )(p_lo, p_hi, b2, W_head, b_head)


def _edge_body(a4, asad, src2, dst2, zrows, zvec, out_hbm,
               as_l, ad_l, den_sl, src_c, dst_c, idx_c, e_c, rows,
               den_sp, out_sp, lsem, dsem, gsem, ssem):
    cid = lax.axis_index("c")
    sid = lax.axis_index("s")

    pltpu.sync_copy(zrows.at[pl.ds(sid * NPT, NPT), :],
                    out_sp.at[pl.ds(sid * NPT, NPT), :])
    pltpu.sync_copy(zvec.at[pl.ds(sid * NPT, NPT)],
                    den_sp.at[pl.ds(sid * NPT, NPT)])
    pltpu.sync_copy(asad.at[0], as_l)
    pltpu.sync_copy(asad.at[1], ad_l)
    plsc.subcore_barrier()

    row_base = sid * PT_ROWS

    pltpu.async_copy(src2.at[pl.ds(row_base, 4), :], src_c.at[0], lsem)
    pltpu.async_copy(dst2.at[pl.ds(row_base, 4), :], dst_c.at[0], lsem)

    def chunk(i, carry):
        p = i % 2
        pltpu.make_async_copy(src2.at[pl.ds(row_base, 4), :],
                              src_c.at[p], lsem).wait()
        pltpu.make_async_copy(dst2.at[pl.ds(row_base, 4), :],
                              dst_c.at[p], lsem).wait()
        for q in range(4):
            for l in range(8):
                sv = src_c[p, q, pl.ds(l * 16, 16)]
                idx_c[p, q, pl.ds(l * 16, 16)] = sv * 4 + cid
        @pl.when(i >= 1)
        def _():
            for q in range(4):
                pltpu.make_async_copy(
                    e_c.at[0, 0], den_sp.at[pl.ds(0, 128)], dsem).wait()
                pltpu.make_async_copy(
                    rows.at[0, pl.ds(0, 128), :],
                    out_sp.at[pl.ds(0, 128), :], ssem).wait()
        gd = []
        for q in range(4):
            gd.append(pltpu.async_copy(
                a4.at[idx_c.at[p, q]],
                rows.at[p, pl.ds(q * 128, 128), :], gsem))
        @pl.when(i < P1_CH - 1)
        def _():
            r1 = row_base + (i + 1) * 4
            pn = (i + 1) % 2
            pltpu.async_copy(src2.at[pl.ds(r1, 4), :], src_c.at[pn], lsem)
            pltpu.async_copy(dst2.at[pl.ds(r1, 4), :], dst_c.at[pn], lsem)
        for q in range(4):
            for l in range(8):
                sv = src_c[p, q, pl.ds(l * 16, 16)]
                dv = dst_c[p, q, pl.ds(l * 16, 16)]
                al = plsc.load_gather(as_l, [sv]) \
                    + plsc.load_gather(ad_l, [dv])
                e_c[p, q, pl.ds(l * 16, 16)] = jnp.exp(_lrelu(al, NEG_G))
        for q in range(4):
            pltpu.async_copy(e_c.at[p, q], den_sp.at[dst_c.at[p, q]],
                             dsem, add=True)
        for d in gd:
            d.wait()
        for q in range(4):
            for l in range(8):
                ev = e_c[p, q, pl.ds(l * 16, 16)]
                for t in range(16):
                    ei = q * 128 + l * 16 + t
                    es = _splat(ev, t)
                    rows[p, ei, pl.ds(0, 16)] = \
                        rows[p, ei, pl.ds(0, 16)] * es
                    rows[p, ei, pl.ds(16, 16)] = \
                        rows[p, ei, pl.ds(16, 16)] * es
        for q in range(4):
            pltpu.async_copy(rows.at[p, pl.ds(q * 128, 128), :],
                             out_sp.at[dst_c.at[p, q]], ssem, add=True)
        return carry

    lax.fori_loop(0, P1_CH, chunk, 0)
    for q in range(4):
        pltpu.make_async_copy(
            e_c.at[0, 0], den_sp.at[pl.ds(0, 128)], dsem).wait()
        pltpu.make_async_copy(
            rows.at[0, pl.ds(0, 128), :],
            out_sp.at[pl.ds(0, 128), :], ssem).wait()
    plsc.subcore_barrier()

    n0 = sid * NPT
    pltpu.sync_copy(out_sp.at[pl.ds(n0, NPT), :],
                    rows.at[0, pl.ds(0, NPT), :])
    pltpu.sync_copy(den_sp.at[pl.ds(n0, NPT)], den_sl)

    def ngroup(g, carry):
        dv = den_sl[pl.ds(g * 16, 16)]
        inv = 1.0 / (dv + 1e-16)
        for t in range(16):
            iv = _splat(inv, t)
            ri = g * 16 + t
            rows[0, ri, pl.ds(0, 16)] = rows[0, ri, pl.ds(0, 16)] * iv
            rows[0, ri, pl.ds(16, 16)] = rows[0, ri, pl.ds(16, 16)] * iv
        return carry

    lax.fori_loop(0, NPT // 16, ngroup, 0)
    pltpu.sync_copy(rows.at[0, pl.ds(0, NPT), :],
                    out_hbm.at[cid, pl.ds(n0, NPT), :])


@functools.cache
def _edge_layer_fn():
  return functools.partial(
    pl.kernel,
    out_type=jax.ShapeDtypeStruct((2, NP, 32), _f32),
    mesh=plsc.VectorSubcoreMesh(core_axis_name="c", subcore_axis_name="s",
                                num_cores=2, num_subcores=NSUB),
    scratch_types=[
        pltpu.VMEM((NP,), _f32),
        pltpu.VMEM((NP,), _f32),
        pltpu.VMEM((NPT,), _f32),
        pltpu.VMEM((2, 4, 128), _i32),
        pltpu.VMEM((2, 4, 128), _i32),
        pltpu.VMEM((2, 4, 128), _i32),
        pltpu.VMEM((2, 4, 128), _f32),
        pltpu.VMEM((2, 1024, 32), _f32),
        pltpu.VMEM_SHARED((NP,), _f32),
        pltpu.VMEM_SHARED((NP, 32), _f32),
        pltpu.SemaphoreType.DMA,
        pltpu.SemaphoreType.DMA,
        pltpu.SemaphoreType.DMA,
        pltpu.SemaphoreType.DMA,
    ],
    compiler_params=pltpu.CompilerParams(needs_layout_passes=False,
                                         use_tc_tiling_on_sc=False),
  )(_edge_body)


def _edge_layer(a4, asad, src2, dst2, zrows, zvec):
    return _edge_layer_fn()(a4, asad, src2, dst2, zrows, zvec)


L2CAP = PT_ROWS * 128


def _edge2_body(a4, asad, src2, dst2, fi_hbm, out_hbm,
                as_l, ad_l, slot_l, den_l, fi_v, sg_v,
                src_c, dst_c, hsrc, hslot, he, hw_idx, rows16,
                den_sp, acc_sp, lsem, dsem, gsem, ssem):
    cid = lax.axis_index("c")
    sid = lax.axis_index("s")

    def memset_chunk(i, carry):
        slot_l[pl.ds(i * 16, 16)] = jnp.full((16,), 16, _i32)
        return carry
    lax.fori_loop(0, NP // 16, memset_chunk, 0)
    fi_v[pl.ds(16, 16)] = jnp.zeros((16,), _i32)
    pltpu.sync_copy(fi_hbm, fi_v.at[pl.ds(0, 16)])
    lanes = lax.iota(_i32, 16)
    plsc.store_scatter(slot_l, [fi_v[pl.ds(0, 16)]], lanes)

    def hfill_chunk(i, carry):
        hsrc[pl.ds(i * 16, 16)] = jnp.zeros((16,), _i32)
        hslot[pl.ds(i * 16, 16)] = jnp.full((16,), 16, _i32)
        return carry
    lax.fori_loop(0, L2CAP // 16, hfill_chunk, 0)

    @pl.when(sid == 0)
    def _():
        z16 = jnp.zeros((16,), _f32)
        den_l[pl.ds(0, 16)] = z16
        den_l[pl.ds(16, 16)] = z16
        pltpu.sync_copy(den_l.at[pl.ds(0, 32)], den_sp)
        for r in range(16):
            rows16[r, pl.ds(0, 16)] = z16
            rows16[r, pl.ds(16, 16)] = z16
        pltpu.sync_copy(rows16, acc_sp.at[pl.ds(0, 16), :])
        pltpu.sync_copy(rows16, acc_sp.at[pl.ds(16, 16), :])

    pltpu.sync_copy(asad.at[0], as_l)
    pltpu.sync_copy(asad.at[1], ad_l)
    plsc.subcore_barrier()

    row_base = sid * PT_ROWS
    pltpu.async_copy(src2.at[pl.ds(row_base, 4), :], src_c.at[0], lsem)
    pltpu.async_copy(dst2.at[pl.ds(row_base, 4), :], dst_c.at[0], lsem)

    def p1_chunk(i, cnt):
        p = i % 2
        pltpu.make_async_copy(src2.at[pl.ds(row_base, 4), :],
                              src_c.at[p], lsem).wait()
        pltpu.make_async_copy(dst2.at[pl.ds(row_base, 4), :],
                              dst_c.at[p], lsem).wait()
        @pl.when(i < P1_CH - 1)
        def _():
            r1 = row_base + (i + 1) * 4
            pn = (i + 1) % 2
            pltpu.async_copy(src2.at[pl.ds(r1, 4), :], src_c.at[pn], lsem)
            pltpu.async_copy(dst2.at[pl.ds(r1, 4), :], dst_c.at[pn], lsem)
        for q in range(4):
            for l in range(8):
                sv = src_c[p, q, pl.ds(l * 16, 16)]
                dv = dst_c[p, q, pl.ds(l * 16, 16)]
                slv = plsc.load_gather(slot_l, [dv])
                hit = slv < 16
                plsc.store_compressed(hsrc.at[pl.ds(cnt, 16)], sv,
                                      mask=hit)
                plsc.store_compressed(hslot.at[pl.ds(cnt, 16)], slv,
                                      mask=hit)
                cnt = cnt + jnp.sum(hit.astype(_i32))
        return cnt

    cnt = lax.fori_loop(0, P1_CH, p1_chunk, 0)
    plsc.subcore_barrier()

    ngroups = (cnt + 15) >> 4

    def p2_group(g, carry):
        b = g * 16
        sv = hsrc[pl.ds(b, 16)]
        slv = hslot[pl.ds(b, 16)]
        hw_idx[0, pl.ds(0, 16)] = sv * 4 + cid
        gd = pltpu.async_copy(a4.at[hw_idx.at[0]], rows16, gsem)
        dvv = plsc.load_gather(fi_v, [slv])
        al = plsc.load_gather(as_l, [sv]) + plsc.load_gather(ad_l, [dvv])
        ev = jnp.exp(_lrelu(al, NEG_G))
        he[pl.ds(0, 16)] = ev
        hw_idx[1, pl.ds(0, 16)] = slv
        pltpu.async_copy(he.at[pl.ds(0, 16)],
                         den_sp.at[hw_idx.at[1]], dsem, add=True).wait()
        gd.wait()
        for t in range(16):
            es = _splat(ev, t)
            rows16[t, pl.ds(0, 16)] = rows16[t, pl.ds(0, 16)] * es
            rows16[t, pl.ds(16, 16)] = rows16[t, pl.ds(16, 16)] * es
        pltpu.async_copy(rows16, acc_sp.at[hw_idx.at[1]],
                         ssem, add=True).wait()
        return carry

    lax.fori_loop(0, ngroups, p2_group, 0)
    plsc.subcore_barrier()

    @pl.when(sid == 0)
    def _():
        pltpu.sync_copy(den_sp, den_l.at[pl.ds(0, 32)])
        sg = plsc.load_gather(slot_l, [fi_v[pl.ds(0, 16)]])
        sg_v[pl.ds(0, 16)] = sg
        pltpu.async_copy(acc_sp.at[sg_v], rows16, gsem).wait()
        inv = 1.0 / (plsc.load_gather(den_l, [sg]) + 1e-16)
        for t in range(16):
            iv = _splat(inv, t)
            rows16[t, pl.ds(0, 16)] = rows16[t, pl.ds(0, 16)] * iv
            rows16[t, pl.ds(16, 16)] = rows16[t, pl.ds(16, 16)] * iv
        pltpu.sync_copy(rows16, out_hbm.at[cid])


@functools.cache
def _edge_layer2_fn():
  return functools.partial(
    pl.kernel,
    out_type=jax.ShapeDtypeStruct((2, G, 32), _f32),
    mesh=plsc.VectorSubcoreMesh(core_axis_name="c", subcore_axis_name="s",
                                num_cores=2, num_subcores=NSUB),
    scratch_types=[
        pltpu.VMEM((NP,), _f32),
        pltpu.VMEM((NP,), _f32),
        pltpu.VMEM((NP,), _i32),
        pltpu.VMEM((32,), _f32),
        pltpu.VMEM((32,), _i32),
        pltpu.VMEM((16,), _i32),
        pltpu.VMEM((2, 4, 128), _i32),
        pltpu.VMEM((2, 4, 128), _i32),
        pltpu.VMEM((L2CAP,), _i32),
        pltpu.VMEM((L2CAP,), _i32),
        pltpu.VMEM((16,), _f32),
        pltpu.VMEM((2, 16), _i32),
        pltpu.VMEM((16, 32), _f32),
        pltpu.VMEM_SHARED((32,), _f32),
        pltpu.VMEM_SHARED((32, 32), _f32),
        pltpu.SemaphoreType.DMA,
        pltpu.SemaphoreType.DMA,
        pltpu.SemaphoreType.DMA,
        pltpu.SemaphoreType.DMA,
    ],
    compiler_params=pltpu.CompilerParams(needs_layout_passes=False,
                                         use_tc_tiling_on_sc=False),
  )(_edge2_body)


def _edge_layer2(a4, asad, src2, dst2, fi16):
    return _edge_layer2_fn()(a4, asad, src2, dst2, fi16)


def kernel(x, edge_index, batch, W_emb, b_emb, lin1, a1s, a1d, b1,
           lin2, a2s, a2d, b2, W_head, b_head):
    xp = jnp.pad(x, ((0, NP - N), (0, 0)))
    batch_p = jnp.pad(batch, (0, NP - N), constant_values=G)

    loop = jnp.arange(N, dtype=_i32)
    npad = PAD_E - ET
    src = jnp.concatenate(
        [edge_index[0], loop, jnp.zeros((npad,), _i32)])
    dst = jnp.concatenate(
        [edge_index[1], loop,
         N + (jnp.arange(npad, dtype=_i32) % (NP - N))])
    src2 = src.reshape(EROWS, 128)
    dst2 = dst.reshape(EROWS, 128)

    z6 = jnp.zeros((6, HID), _f32)
    l1cat = jnp.concatenate(
        [lin1, a1s[:, None], a1d[:, None], jnp.zeros((HID, 62), _f32)], 1)
    l2cat = jnp.concatenate(
        [lin2, a2s[:, None], a2d[:, None], jnp.zeros((HID, 62), _f32)], 1)
    aa1 = jnp.concatenate([a1s[None], a1d[None], z6], 0)
    aa2 = jnp.concatenate([a2s[None], a2d[None], z6], 0)

    zrows = jnp.zeros((NP, 32), _f32)
    zvec = jnp.zeros((NP,), _f32)

    cnt, fi = _counts(batch_p.reshape(80, 128))

    a1_, asad1 = _dense1(xp, W_emb, b_emb.reshape(1, HID), l1cat, aa1)
    p1 = _edge_layer(a1_.reshape(NP * 4, 32), asad1, src2, dst2, zrows, zvec)

    a2_, asad2 = _dense2(p1[0], p1[1], b1.reshape(1, HID), l2cat, aa2)
    p2 = _edge_layer2(a2_.reshape(NP * 4, 32), asad2, src2, dst2, fi[:, 0])

    out = _readout(p2[0], p2[1], b2.reshape(1, HID),
                   W_head, b_head.reshape(1, OUT))
    num_nodes = cnt[:, :1]
    return (out, num_nodes)
